# baseline (speedup 1.0000x reference)
import functools

import jax
import jax.numpy as jnp
from jax import lax
from jax.experimental import pallas as pl
from jax.experimental.pallas import tpu as pltpu
from jax.experimental.pallas import tpu_sc as plsc

_EPS = 1e-5
_L = 16


def _pack_pairs(f):
    ah = lax.bitcast_convert_type(f[:, :64].astype(jnp.bfloat16),
                                  jnp.uint16).astype(jnp.uint32)
    al = lax.bitcast_convert_type(f[:, 64:].astype(jnp.bfloat16),
                                  jnp.uint16).astype(jnp.uint32)
    return lax.bitcast_convert_type((ah << 16) | al, jnp.float32)


def _node_prologue(n_real, x_ref, wn_ref, bn_ref, gn_ref, betan_ref, w1_ref,
                   w2_ref, nf_ref, a_ref, b_ref):
    x = x_ref[...]
    n_pad = x.shape[0]
    u = jnp.maximum(
        jnp.dot(x, wn_ref[...], preferred_element_type=jnp.float32) + bn_ref[...],
        0.0)
    rb = jnp.maximum(bn_ref[...], 0.0)
    extra = jnp.float32(n_pad - n_real)
    mean = (jnp.sum(u, axis=0, keepdims=True) - extra * rb) / n_real
    var = ((jnp.sum(u * u, axis=0, keepdims=True) - extra * rb * rb) / n_real
           - mean * mean)
    scale = gn_ref[...] * lax.rsqrt(var + _EPS)
    nf_ref[...] = u * scale + (betan_ref[...] - mean * scale)
    a_ref[...] = jnp.dot(x, w1_ref[...], preferred_element_type=jnp.float32)
    b_ref[...] = jnp.dot(x, w2_ref[...], preferred_element_type=jnp.float32)


def _edge_stats(ea_ref, we_ref, be_ref, zs_ref):
    i = pl.program_id(0)
    z = jnp.maximum(
        jnp.dot(ea_ref[...], we_ref[...], preferred_element_type=jnp.float32)
        + be_ref[...], 0.0)
    blk = jnp.concatenate(
        [jnp.sum(z, axis=0, keepdims=True),
         jnp.sum(z * z, axis=0, keepdims=True)], axis=0)

    @pl.when(i == 0)
    def _():
        zs_ref[...] = blk

    @pl.when(i > 0)
    def _():
        zs_ref[...] = zs_ref[...] + blk


def _edge_msg(n_edges, zs_ref, we_ref, be_ref, ge_ref, betae_ref, w3_ref,
              bmsg_ref, ea_ref, p_ref):
    mean = zs_ref[0:1, :] / n_edges
    var = zs_ref[1:2, :] / n_edges - mean * mean
    s = ge_ref[...] * lax.rsqrt(var + _EPS)
    t = betae_ref[...] - mean * s
    z = jnp.maximum(
        jnp.dot(ea_ref[...], we_ref[...], preferred_element_type=jnp.float32)
        + be_ref[...], 0.0)
    const = jnp.dot(t, w3_ref[...], preferred_element_type=jnp.float32) + bmsg_ref[...]
    p_ref[...] = _pack_pairs(
        jnp.dot(z * s, w3_ref[...], preferred_element_type=jnp.float32)
        + const)


def _epilogue(n_edges, ys_ref, gm_ref, betam_ref, s_ref, d_ref, nf_ref,
              out_ref):
    ysum = jnp.sum(ys_ref[:, 0, :], axis=0, keepdims=True)
    ysq = jnp.sum(ys_ref[:, 1, :], axis=0, keepdims=True)
    mean = ysum / n_edges
    var = ysq / n_edges - mean * mean
    s_m = gm_ref[...] * lax.rsqrt(var + _EPS)
    t_m = betam_ref[...] - mean * s_m
    stot = s_ref[0] + s_ref[1]
    deg = d_ref[0] + d_ref[1]
    out_ref[...] = stot * s_m[None] + deg * t_m[None] + nf_ref[...]


def _make_sc_kernel(n_pad, n_edges, nc, ns):
    nw = nc * ns
    epw = n_edges // nw
    ch = 40
    nchunk = epw // ch
    rpt = n_pad // ns
    mesh = plsc.VectorSubcoreMesh(core_axis_name="c", subcore_axis_name="s",
                                  num_cores=nc, num_subcores=ns)

    @functools.partial(
        pl.kernel,
        out_type=(
            jax.ShapeDtypeStruct((nc, n_pad, 128), jnp.float32),
            jax.ShapeDtypeStruct((nw, 2, 128), jnp.float32),
        ),
        mesh=mesh,
        scratch_types=(
            pltpu.VMEM((2, ch), jnp.int32),
            pltpu.VMEM((2, ch), jnp.int32),
            pltpu.VMEM((2, ch), jnp.int32),
            pltpu.VMEM((2, ch, 128), jnp.float32),
            pltpu.VMEM((2, ch, 128), jnp.float32),
            pltpu.VMEM((2, ch, 64), jnp.float32),
            pltpu.VMEM((2, ch, 128), jnp.float32),
            pltpu.VMEM((2, 128), jnp.float32),
            pltpu.VMEM_SHARED((n_pad, 128), jnp.float32),
            pltpu.SemaphoreType.DMA,
            pltpu.SemaphoreType.DMA,
            pltpu.SemaphoreType.DMA,
            pltpu.SemaphoreType.DMA,
            pltpu.SemaphoreType.DMA,
            pltpu.SemaphoreType.DMA,
            pltpu.SemaphoreType.DMA,
            pltpu.SemaphoreType.DMA,
        ),
    )
    def sc_kernel(a_hbm, b_hbm, p_hbm, row_hbm, col_hbm, ztab_hbm,
                  s_out, ystats_out,
                  ridx, cidx, scidx, abuf, bbuf, pbuf, ybuf, stats,
                  s_tab, gsem0, gsem1, isem0, isem1, csem0, csem1,
                  ssem0, ssem1):
        c = lax.axis_index("c")
        s = lax.axis_index("s")
        wid = s * nc + c
        base = wid * epw
        gsems = (gsem0, gsem1)
        isems = (isem0, isem1)
        csems = (csem0, csem1)
        ssems = (ssem0, ssem1)

        pltpu.sync_copy(ztab_hbm.at[pl.ds(s * rpt, rpt)],
                        s_tab.at[pl.ds(s * rpt, rpt)])
        zero = jnp.zeros((_L,), jnp.float32)
        for j in range(8):
            stats[0, pl.ds(j * _L, _L)] = zero
            stats[1, pl.ds(j * _L, _L)] = zero
        plsc.subcore_barrier()

        def fire_i(g, b):
            off = base + g * ch
            pltpu.async_copy(row_hbm.at[pl.ds(off, ch)], ridx.at[b], isems[b])
            pltpu.async_copy(col_hbm.at[pl.ds(off, ch)], cidx.at[b], isems[b])

        def wait_i(b):
            pltpu.make_async_copy(row_hbm.at[pl.ds(0, ch)], ridx.at[b],
                                  isems[b]).wait()
            pltpu.make_async_copy(col_hbm.at[pl.ds(0, ch)], cidx.at[b],
                                  isems[b]).wait()

        def fire_g(g, b):
            off = base + g * ch
            pltpu.async_copy(a_hbm.at[ridx.at[b]], abuf.at[b], gsems[b])
            pltpu.async_copy(b_hbm.at[cidx.at[b]], bbuf.at[b], gsems[b])
            pltpu.async_copy(p_hbm.at[pl.ds(off, ch)], pbuf.at[b], gsems[b])

        def wait_g(b):
            pltpu.make_async_copy(a_hbm.at[ridx.at[b]], abuf.at[b],
                                  gsems[b]).wait()
            pltpu.make_async_copy(b_hbm.at[cidx.at[b]], bbuf.at[b],
                                  gsems[b]).wait()
            pltpu.make_async_copy(p_hbm.at[pl.ds(0, ch)], pbuf.at[b],
                                  gsems[b]).wait()

        def fire_c(g, b):
            off = base + g * ch
            pltpu.async_copy(col_hbm.at[pl.ds(off, ch)], scidx.at[b], csems[b])

        def wait_c(b):
            pltpu.make_async_copy(col_hbm.at[pl.ds(0, ch)], scidx.at[b],
                                  csems[b]).wait()

        def wait_s(b):
            pltpu.make_async_copy(ybuf.at[b], s_tab.at[scidx.at[b]],
                                  ssems[b]).wait()

        himask = jnp.full((_L,), -65536, jnp.int32)

        def unpack2(v):
            i = lax.bitcast_convert_type(v, jnp.int32)
            hi = lax.bitcast_convert_type(lax.bitwise_and(i, himask),
                                          jnp.float32)
            lo = lax.bitcast_convert_type(lax.shift_left(i, 16), jnp.float32)
            return hi, lo

        def compute(b):
            def row_body(r, acc):
                out = list(acc)
                for m in range(4):
                    ph, pol = unpack2(pbuf[b, r, pl.ds(m * _L, _L)])
                    s0 = pl.ds(32 * m, _L)
                    s1 = pl.ds(32 * m + _L, _L)
                    yh = jnp.maximum(abuf[b, r, s0] + bbuf[b, r, s0] + ph,
                                     0.0)
                    yl = jnp.maximum(abuf[b, r, s1] + bbuf[b, r, s1] + pol,
                                     0.0)
                    ybuf[b, r, s0] = yh
                    ybuf[b, r, s1] = yl
                    out[4 * m] = out[4 * m] + yh
                    out[4 * m + 1] = out[4 * m + 1] + yh * yh
                    out[4 * m + 2] = out[4 * m + 2] + yl
                    out[4 * m + 3] = out[4 * m + 3] + yl * yl
                return tuple(out)

            acc0 = tuple(jnp.zeros((_L,), jnp.float32) for _ in range(16))
            acc = lax.fori_loop(0, ch, row_body, acc0)
            for m in range(4):
                s0 = pl.ds(32 * m, _L)
                s1 = pl.ds(32 * m + _L, _L)
                stats[0, s0] = stats[0, s0] + acc[4 * m]
                stats[1, s0] = stats[1, s0] + acc[4 * m + 1]
                stats[0, s1] = stats[0, s1] + acc[4 * m + 2]
                stats[1, s1] = stats[1, s1] + acc[4 * m + 3]
            wait_c(b)
            pltpu.async_copy(ybuf.at[b], s_tab.at[scidx.at[b]], ssems[b],
                             add=True)

        fire_i(0, 0)
        fire_i(1, 1)
        wait_i(0)
        fire_g(0, 0)

        def pair_body(t, carry):
            for b in range(2):
                g = 2 * t + b
                nb = 1 - b

                @pl.when(jnp.logical_and(g >= 2, g < nchunk))
                def _():
                    wait_s(b)

                @pl.when(g < nchunk)
                def _():
                    fire_c(g, b)

                @pl.when(g + 1 < nchunk)
                def _():
                    wait_i(nb)
                    fire_g(g + 1, nb)

                @pl.when(g < nchunk)
                def _():
                    wait_g(b)
                    compute(b)

                @pl.when(g + 2 < nchunk)
                def _():
                    fire_i(g + 2, b)
            return carry

        lax.fori_loop(0, (nchunk + 1) // 2, pair_body, 0)
        wait_s(0)
        wait_s(1)

        pltpu.sync_copy(stats, ystats_out.at[wid])

        plsc.subcore_barrier()
        pltpu.sync_copy(s_tab.at[pl.ds(s * rpt, rpt)],
                        s_out.at[c, pl.ds(s * rpt, rpt)])

    return sc_kernel


def _make_deg_kernel(n_pad, n_edges, nc, ns):
    nw = nc * ns
    epw = n_edges // nw
    ch = 80
    nchunk = epw // ch
    rpt = n_pad // ns
    mesh = plsc.VectorSubcoreMesh(core_axis_name="c", subcore_axis_name="s",
                                  num_cores=nc, num_subcores=ns)

    @functools.partial(
        pl.kernel,
        out_type=jax.ShapeDtypeStruct((nc, n_pad, 128), jnp.float32),
        mesh=mesh,
        scratch_types=(
            pltpu.VMEM((ch,), jnp.int32),
            pltpu.VMEM((ch, 128), jnp.float32),
            pltpu.VMEM_SHARED((n_pad, 128), jnp.float32),
        ),
    )
    def deg_kernel(col_hbm, ztab_hbm, ones_hbm, d_out, cidx, ones_v, d_tab):
        c = lax.axis_index("c")
        s = lax.axis_index("s")
        wid = s * nc + c
        base = wid * epw

        pltpu.sync_copy(ztab_hbm.at[pl.ds(s * rpt, rpt)],
                        d_tab.at[pl.ds(s * rpt, rpt)])
        pltpu.sync_copy(ones_hbm, ones_v)
        plsc.subcore_barrier()

        def chunk_body(i, carry):
            pltpu.sync_copy(col_hbm.at[pl.ds(base + i * ch, ch)], cidx)
            pltpu.sync_copy(ones_v, d_tab.at[cidx], add=True)
            return carry

        lax.fori_loop(0, nchunk, chunk_body, 0)

        plsc.subcore_barrier()
        pltpu.sync_copy(d_tab.at[pl.ds(s * rpt, rpt)],
                        d_out.at[c, pl.ds(s * rpt, rpt)])

    return deg_kernel


def kernel(x, edge_attr, W_node, b_node, g_node, beta_node,
           W_edge, b_edge, g_edge, beta_edge,
           W_msg, b_msg, g_msg, beta_msg, edge_index):
    n_nodes, in_ch = x.shape
    n_edges = edge_attr.shape[0]
    out_ch = W_node.shape[1]

    hi_cols = [32 * m + i for m in range(4) for i in range(16)]
    lo_cols = [32 * m + 16 + i for m in range(4) for i in range(16)]
    perm = jnp.array(hi_cols + lo_cols, jnp.int32)
    w1 = W_msg[:in_ch]
    w2 = W_msg[in_ch:2 * in_ch]
    w3 = W_msg[2 * in_ch:][:, perm]
    bn = b_node.reshape(1, -1)
    gn = g_node.reshape(1, -1)
    betan = beta_node.reshape(1, -1)
    be = b_edge.reshape(1, -1)
    ge = g_edge.reshape(1, -1)
    betae = beta_edge.reshape(1, -1)
    bm = b_msg[perm].reshape(1, -1)
    gm = g_msg.reshape(1, -1)
    betam = beta_msg.reshape(1, -1)

    n_pad = ((n_nodes + 1023) // 1024) * 1024
    x_pad = jnp.pad(x, ((0, n_pad - n_nodes), (0, 0)))

    nf, a_tab, b_tab = pl.pallas_call(
        functools.partial(_node_prologue, n_nodes),
        out_shape=(
            jax.ShapeDtypeStruct((n_pad, out_ch), jnp.float32),
            jax.ShapeDtypeStruct((n_pad, out_ch), jnp.float32),
            jax.ShapeDtypeStruct((n_pad, out_ch), jnp.float32),
        ),
    )(x_pad, W_node, bn, gn, betan, w1, w2)

    eblk1 = 20000
    g1 = n_edges // eblk1
    zstats = pl.pallas_call(
        _edge_stats,
        grid=(g1,),
        in_specs=[
            pl.BlockSpec((eblk1, edge_attr.shape[1]), lambda i: (i, 0)),
            pl.BlockSpec(W_edge.shape, lambda i: (0, 0)),
            pl.BlockSpec((1, out_ch), lambda i: (0, 0)),
        ],
        out_specs=pl.BlockSpec((2, out_ch), lambda i: (0, 0)),
        out_shape=jax.ShapeDtypeStruct((2, out_ch), jnp.float32),
    )(edge_attr, W_edge, be)

    eblk2 = 8000
    g2 = n_edges // eblk2
    p_tab = pl.pallas_call(
        functools.partial(_edge_msg, n_edges),
        grid=(g2,),
        in_specs=[
            pl.BlockSpec((2, out_ch), lambda i: (0, 0)),
            pl.BlockSpec(W_edge.shape, lambda i: (0, 0)),
            pl.BlockSpec((1, out_ch), lambda i: (0, 0)),
            pl.BlockSpec((1, out_ch), lambda i: (0, 0)),
            pl.BlockSpec((1, out_ch), lambda i: (0, 0)),
            pl.BlockSpec(w3.shape, lambda i: (0, 0)),
            pl.BlockSpec((1, out_ch), lambda i: (0, 0)),
            pl.BlockSpec((eblk2, edge_attr.shape[1]), lambda i: (i, 0)),
        ],
        out_specs=pl.BlockSpec((eblk2, out_ch // 2), lambda i: (i, 0)),
        out_shape=jax.ShapeDtypeStruct((n_edges, out_ch // 2), jnp.float32),
    )(zstats, W_edge, be, ge, betae, w3, bm, edge_attr)

    info = plsc.get_sparse_core_info()
    nc, ns = info.num_cores, info.num_subcores
    nw = nc * ns
    row = edge_index[0]
    col = edge_index[1]
    zeros_tab = jnp.zeros((n_pad, 128), jnp.float32)
    ones_rows = jnp.ones((80, 128), jnp.float32)
    deg_k = _make_deg_kernel(n_pad, n_edges, nc, ns)
    d_part = deg_k(col, zeros_tab, ones_rows)
    sc = _make_sc_kernel(n_pad, n_edges, nc, ns)
    s_part, ystats = sc(a_tab, b_tab, p_tab, row, col, zeros_tab)

    dgrp = n_pad // 128
    s_part4 = s_part.reshape(nc, dgrp, 128, 128)
    d_part4 = d_part.reshape(nc, dgrp, 128, 128)
    nf4 = nf.reshape(dgrp, 128, 128)
    gblk = 16
    g3 = dgrp // gblk
    out = pl.pallas_call(
        functools.partial(_epilogue, n_edges),
        grid=(g3,),
        in_specs=[
            pl.BlockSpec((nw, 2, out_ch), lambda i: (0, 0, 0)),
            pl.BlockSpec((1, out_ch), lambda i: (0, 0)),
            pl.BlockSpec((1, out_ch), lambda i: (0, 0)),
            pl.BlockSpec((nc, gblk, 128, 128), lambda i: (0, i, 0, 0)),
            pl.BlockSpec((nc, gblk, 128, 128), lambda i: (0, i, 0, 0)),
            pl.BlockSpec((gblk, 128, 128), lambda i: (i, 0, 0)),
        ],
        out_specs=pl.BlockSpec((gblk, 128, 128), lambda i: (i, 0, 0)),
        out_shape=jax.ShapeDtypeStruct((dgrp, 128, 128), jnp.float32),
    )(ystats, gm, betam, s_part4, d_part4, nf4)
    return out.reshape(n_pad, out_ch)[:n_nodes]


# speedup vs baseline: 1.0819x; 1.0819x over previous
import functools

import jax
import jax.numpy as jnp
from jax import lax
from jax.experimental import pallas as pl
from jax.experimental.pallas import tpu as pltpu
from jax.experimental.pallas import tpu_sc as plsc

_EPS = 1e-5
_L = 16


def _pack_pairs(f):
    u = lax.bitcast_convert_type(f, jnp.uint32) + jnp.uint32(0x8000)
    hi = u[:, :64] & jnp.uint32(0xFFFF0000)
    lo = u[:, 64:] >> 16
    return lax.bitcast_convert_type(hi | lo, jnp.float32)


def _node_prologue(n_real, x_ref, wn_ref, bn_ref, gn_ref, betan_ref, w1_ref,
                   w2_ref, nf_ref, a_ref, b_ref):
    x = x_ref[...]
    n_pad = x.shape[0]
    u = jnp.maximum(
        jnp.dot(x, wn_ref[...], preferred_element_type=jnp.float32) + bn_ref[...],
        0.0)
    rb = jnp.maximum(bn_ref[...], 0.0)
    extra = jnp.float32(n_pad - n_real)
    mean = (jnp.sum(u, axis=0, keepdims=True) - extra * rb) / n_real
    var = ((jnp.sum(u * u, axis=0, keepdims=True) - extra * rb * rb) / n_real
           - mean * mean)
    scale = gn_ref[...] * lax.rsqrt(var + _EPS)
    nf_ref[...] = u * scale + (betan_ref[...] - mean * scale)
    a_ref[...] = jnp.dot(x, w1_ref[...], preferred_element_type=jnp.float32)
    b_ref[...] = jnp.dot(x, w2_ref[...], preferred_element_type=jnp.float32)


def _edge_stats(ea_ref, we_ref, be_ref, zs_ref):
    i = pl.program_id(0)
    z = jnp.maximum(
        jnp.dot(ea_ref[...], we_ref[...], preferred_element_type=jnp.float32)
        + be_ref[...], 0.0)
    blk = jnp.concatenate(
        [jnp.sum(z, axis=0, keepdims=True),
         jnp.sum(z * z, axis=0, keepdims=True)], axis=0)

    @pl.when(i == 0)
    def _():
        zs_ref[...] = blk

    @pl.when(i > 0)
    def _():
        zs_ref[...] = zs_ref[...] + blk


def _edge_msg(n_edges, zs_ref, we_ref, be_ref, ge_ref, betae_ref, w3_ref,
              bmsg_ref, ea_ref, p_ref):
    mean = zs_ref[0:1, :] / n_edges
    var = zs_ref[1:2, :] / n_edges - mean * mean
    s = ge_ref[...] * lax.rsqrt(var + _EPS)
    t = betae_ref[...] - mean * s
    z = jnp.maximum(
        jnp.dot(ea_ref[...], we_ref[...], preferred_element_type=jnp.float32)
        + be_ref[...], 0.0)
    const = jnp.dot(t, w3_ref[...], preferred_element_type=jnp.float32) + bmsg_ref[...]
    p_ref[...] = _pack_pairs(
        jnp.dot(z * s, w3_ref[...], preferred_element_type=jnp.float32)
        + const)


def _epilogue(n_edges, ys_ref, gm_ref, betam_ref, s_ref, d_ref, nf_ref,
              out_ref):
    ysum = jnp.sum(ys_ref[:, 0, :], axis=0, keepdims=True)
    ysq = jnp.sum(ys_ref[:, 1, :], axis=0, keepdims=True)
    mean = ysum / n_edges
    var = ysq / n_edges - mean * mean
    s_m = gm_ref[...] * lax.rsqrt(var + _EPS)
    t_m = betam_ref[...] - mean * s_m
    stot = s_ref[0] + s_ref[1]
    deg = d_ref[0] + d_ref[1]
    out_ref[...] = stot * s_m[None] + deg * t_m[None] + nf_ref[...]


def _make_sc_kernel(n_pad, n_edges, nc, ns):
    nw = nc * ns
    epw = n_edges // nw
    ch = 40
    nchunk = epw // ch
    rpt = n_pad // ns
    mesh = plsc.VectorSubcoreMesh(core_axis_name="c", subcore_axis_name="s",
                                  num_cores=nc, num_subcores=ns)

    @functools.partial(
        pl.kernel,
        out_type=(
            jax.ShapeDtypeStruct((nc, n_pad, 128), jnp.float32),
            jax.ShapeDtypeStruct((nw, 2, 128), jnp.float32),
        ),
        mesh=mesh,
        scratch_types=(
            pltpu.VMEM((2, ch), jnp.int32),
            pltpu.VMEM((2, ch), jnp.int32),
            pltpu.VMEM((2, ch), jnp.int32),
            pltpu.VMEM((2, ch, 128), jnp.float32),
            pltpu.VMEM((2, ch, 128), jnp.float32),
            pltpu.VMEM((2, ch, 64), jnp.float32),
            pltpu.VMEM((2, ch, 128), jnp.float32),
            pltpu.VMEM((2, 128), jnp.float32),
            pltpu.VMEM_SHARED((n_pad, 128), jnp.float32),
            pltpu.SemaphoreType.DMA,
            pltpu.SemaphoreType.DMA,
            pltpu.SemaphoreType.DMA,
            pltpu.SemaphoreType.DMA,
            pltpu.SemaphoreType.DMA,
            pltpu.SemaphoreType.DMA,
            pltpu.SemaphoreType.DMA,
            pltpu.SemaphoreType.DMA,
        ),
    )
    def sc_kernel(a_hbm, b_hbm, p_hbm, row_hbm, col_hbm, ztab_hbm,
                  s_out, ystats_out,
                  ridx, cidx, scidx, abuf, bbuf, pbuf, ybuf, stats,
                  s_tab, gsem0, gsem1, isem0, isem1, csem0, csem1,
                  ssem0, ssem1):
        c = lax.axis_index("c")
        s = lax.axis_index("s")
        wid = s * nc + c
        base = wid * epw
        gsems = (gsem0, gsem1)
        isems = (isem0, isem1)
        csems = (csem0, csem1)
        ssems = (ssem0, ssem1)

        pltpu.sync_copy(ztab_hbm.at[pl.ds(s * rpt, rpt)],
                        s_tab.at[pl.ds(s * rpt, rpt)])
        zero = jnp.zeros((_L,), jnp.float32)
        for j in range(8):
            stats[0, pl.ds(j * _L, _L)] = zero
            stats[1, pl.ds(j * _L, _L)] = zero
        plsc.subcore_barrier()

        def fire_i(g, b):
            off = base + g * ch
            pltpu.async_copy(row_hbm.at[pl.ds(off, ch)], ridx.at[b], isems[b])
            pltpu.async_copy(col_hbm.at[pl.ds(off, ch)], cidx.at[b], isems[b])

        def wait_i(b):
            pltpu.make_async_copy(row_hbm.at[pl.ds(0, ch)], ridx.at[b],
                                  isems[b]).wait()
            pltpu.make_async_copy(col_hbm.at[pl.ds(0, ch)], cidx.at[b],
                                  isems[b]).wait()

        def fire_g(g, b):
            off = base + g * ch
            pltpu.async_copy(a_hbm.at[ridx.at[b]], abuf.at[b], gsems[b])
            pltpu.async_copy(b_hbm.at[cidx.at[b]], bbuf.at[b], gsems[b])
            pltpu.async_copy(p_hbm.at[pl.ds(off, ch)], pbuf.at[b], gsems[b])

        def wait_g(b):
            pltpu.make_async_copy(a_hbm.at[ridx.at[b]], abuf.at[b],
                                  gsems[b]).wait()
            pltpu.make_async_copy(b_hbm.at[cidx.at[b]], bbuf.at[b],
                                  gsems[b]).wait()
            pltpu.make_async_copy(p_hbm.at[pl.ds(0, ch)], pbuf.at[b],
                                  gsems[b]).wait()

        def fire_c(g, b):
            off = base + g * ch
            pltpu.async_copy(col_hbm.at[pl.ds(off, ch)], scidx.at[b], csems[b])

        def wait_c(b):
            pltpu.make_async_copy(col_hbm.at[pl.ds(0, ch)], scidx.at[b],
                                  csems[b]).wait()

        def wait_s(b):
            pltpu.make_async_copy(ybuf.at[b], s_tab.at[scidx.at[b]],
                                  ssems[b]).wait()

        himask = jnp.full((_L,), -65536, jnp.int32)

        def unpack2(v):
            i = lax.bitcast_convert_type(v, jnp.int32)
            hi = lax.bitcast_convert_type(lax.bitwise_and(i, himask),
                                          jnp.float32)
            lo = lax.bitcast_convert_type(lax.shift_left(i, 16), jnp.float32)
            return hi, lo

        def compute(b):
            def row_body(r, acc):
                out = list(acc)
                for m in range(4):
                    ph, pol = unpack2(pbuf[b, r, pl.ds(m * _L, _L)])
                    s0 = pl.ds(32 * m, _L)
                    s1 = pl.ds(32 * m + _L, _L)
                    yh = jnp.maximum(abuf[b, r, s0] + bbuf[b, r, s0] + ph,
                                     0.0)
                    yl = jnp.maximum(abuf[b, r, s1] + bbuf[b, r, s1] + pol,
                                     0.0)
                    ybuf[b, r, s0] = yh
                    ybuf[b, r, s1] = yl
                    out[4 * m] = out[4 * m] + yh
                    out[4 * m + 1] = out[4 * m + 1] + yh * yh
                    out[4 * m + 2] = out[4 * m + 2] + yl
                    out[4 * m + 3] = out[4 * m + 3] + yl * yl
                return tuple(out)

            acc0 = tuple(jnp.zeros((_L,), jnp.float32) for _ in range(16))
            acc = lax.fori_loop(0, ch, row_body, acc0)
            for m in range(4):
                s0 = pl.ds(32 * m, _L)
                s1 = pl.ds(32 * m + _L, _L)
                stats[0, s0] = stats[0, s0] + acc[4 * m]
                stats[1, s0] = stats[1, s0] + acc[4 * m + 1]
                stats[0, s1] = stats[0, s1] + acc[4 * m + 2]
                stats[1, s1] = stats[1, s1] + acc[4 * m + 3]
            wait_c(b)
            pltpu.async_copy(ybuf.at[b], s_tab.at[scidx.at[b]], ssems[b],
                             add=True)

        fire_i(0, 0)
        fire_i(1, 1)
        wait_i(0)
        fire_g(0, 0)

        def pair_body(t, carry):
            for b in range(2):
                g = 2 * t + b
                nb = 1 - b

                @pl.when(jnp.logical_and(g >= 2, g < nchunk))
                def _():
                    wait_s(b)

                @pl.when(g < nchunk)
                def _():
                    fire_c(g, b)

                @pl.when(g + 1 < nchunk)
                def _():
                    wait_i(nb)
                    fire_g(g + 1, nb)

                @pl.when(g < nchunk)
                def _():
                    wait_g(b)
                    compute(b)

                @pl.when(g + 2 < nchunk)
                def _():
                    fire_i(g + 2, b)
            return carry

        lax.fori_loop(0, (nchunk + 1) // 2, pair_body, 0)
        wait_s(0)
        wait_s(1)

        pltpu.sync_copy(stats, ystats_out.at[wid])

        plsc.subcore_barrier()
        pltpu.sync_copy(s_tab.at[pl.ds(s * rpt, rpt)],
                        s_out.at[c, pl.ds(s * rpt, rpt)])

    return sc_kernel


def _make_deg_kernel(n_pad, n_edges, nc, ns):
    nw = nc * ns
    epw = n_edges // nw
    ch = 80
    nchunk = epw // ch
    rpt = n_pad // ns
    mesh = plsc.VectorSubcoreMesh(core_axis_name="c", subcore_axis_name="s",
                                  num_cores=nc, num_subcores=ns)

    @functools.partial(
        pl.kernel,
        out_type=jax.ShapeDtypeStruct((nc, n_pad, 128), jnp.float32),
        mesh=mesh,
        scratch_types=(
            pltpu.VMEM((2, ch), jnp.int32),
            pltpu.VMEM((ch, 128), jnp.float32),
            pltpu.VMEM_SHARED((n_pad, 128), jnp.float32),
            pltpu.SemaphoreType.DMA,
            pltpu.SemaphoreType.DMA,
            pltpu.SemaphoreType.DMA,
            pltpu.SemaphoreType.DMA,
        ),
    )
    def deg_kernel(col_hbm, ztab_hbm, ones_hbm, d_out, cidx, ones_v, d_tab,
                   csem0, csem1, ssem0, ssem1):
        c = lax.axis_index("c")
        s = lax.axis_index("s")
        base = (s * nc + c) * epw
        csems = (csem0, csem1)
        ssems = (ssem0, ssem1)

        pltpu.sync_copy(ztab_hbm.at[pl.ds(s * rpt, rpt)],
                        d_tab.at[pl.ds(s * rpt, rpt)])
        pltpu.sync_copy(ones_hbm, ones_v)
        plsc.subcore_barrier()

        def wait_s(b):
            pltpu.make_async_copy(ones_v, d_tab.at[cidx.at[b]],
                                  ssems[b]).wait()

        def pair_body(t, carry):
            for b in range(2):
                g = 2 * t + b

                @pl.when(jnp.logical_and(g >= 2, g < nchunk))
                def _():
                    wait_s(b)

                @pl.when(g < nchunk)
                def _():
                    pltpu.async_copy(col_hbm.at[pl.ds(base + g * ch, ch)],
                                     cidx.at[b], csems[b])
                    pltpu.make_async_copy(col_hbm.at[pl.ds(0, ch)],
                                          cidx.at[b], csems[b]).wait()
                    pltpu.async_copy(ones_v, d_tab.at[cidx.at[b]], ssems[b],
                                     add=True)
            return carry

        lax.fori_loop(0, (nchunk + 1) // 2, pair_body, 0)
        wait_s(0)
        wait_s(1)

        plsc.subcore_barrier()
        pltpu.sync_copy(d_tab.at[pl.ds(s * rpt, rpt)],
                        d_out.at[c, pl.ds(s * rpt, rpt)])

    return deg_kernel


def kernel(x, edge_attr, W_node, b_node, g_node, beta_node,
           W_edge, b_edge, g_edge, beta_edge,
           W_msg, b_msg, g_msg, beta_msg, edge_index):
    n_nodes, in_ch = x.shape
    n_edges = edge_attr.shape[0]
    out_ch = W_node.shape[1]

    hi_cols = [32 * m + i for m in range(4) for i in range(16)]
    lo_cols = [32 * m + 16 + i for m in range(4) for i in range(16)]
    perm = jnp.array(hi_cols + lo_cols, jnp.int32)
    w1 = W_msg[:in_ch]
    w2 = W_msg[in_ch:2 * in_ch]
    w3 = W_msg[2 * in_ch:][:, perm]
    bn = b_node.reshape(1, -1)
    gn = g_node.reshape(1, -1)
    betan = beta_node.reshape(1, -1)
    be = b_edge.reshape(1, -1)
    ge = g_edge.reshape(1, -1)
    betae = beta_edge.reshape(1, -1)
    bm = b_msg[perm].reshape(1, -1)
    gm = g_msg.reshape(1, -1)
    betam = beta_msg.reshape(1, -1)

    n_pad = ((n_nodes + 1023) // 1024) * 1024
    x_pad = jnp.pad(x, ((0, n_pad - n_nodes), (0, 0)))

    nf, a_tab, b_tab = pl.pallas_call(
        functools.partial(_node_prologue, n_nodes),
        out_shape=(
            jax.ShapeDtypeStruct((n_pad, out_ch), jnp.float32),
            jax.ShapeDtypeStruct((n_pad, out_ch), jnp.float32),
            jax.ShapeDtypeStruct((n_pad, out_ch), jnp.float32),
        ),
    )(x_pad, W_node, bn, gn, betan, w1, w2)

    eblk1 = 20000
    g1 = n_edges // eblk1
    zstats = pl.pallas_call(
        _edge_stats,
        grid=(g1,),
        in_specs=[
            pl.BlockSpec((eblk1, edge_attr.shape[1]), lambda i: (i, 0)),
            pl.BlockSpec(W_edge.shape, lambda i: (0, 0)),
            pl.BlockSpec((1, out_ch), lambda i: (0, 0)),
        ],
        out_specs=pl.BlockSpec((2, out_ch), lambda i: (0, 0)),
        out_shape=jax.ShapeDtypeStruct((2, out_ch), jnp.float32),
    )(edge_attr, W_edge, be)

    eblk2 = 8000
    g2 = n_edges // eblk2
    p_tab = pl.pallas_call(
        functools.partial(_edge_msg, n_edges),
        grid=(g2,),
        in_specs=[
            pl.BlockSpec((2, out_ch), lambda i: (0, 0)),
            pl.BlockSpec(W_edge.shape, lambda i: (0, 0)),
            pl.BlockSpec((1, out_ch), lambda i: (0, 0)),
            pl.BlockSpec((1, out_ch), lambda i: (0, 0)),
            pl.BlockSpec((1, out_ch), lambda i: (0, 0)),
            pl.BlockSpec(w3.shape, lambda i: (0, 0)),
            pl.BlockSpec((1, out_ch), lambda i: (0, 0)),
            pl.BlockSpec((eblk2, edge_attr.shape[1]), lambda i: (i, 0)),
        ],
        out_specs=pl.BlockSpec((eblk2, out_ch // 2), lambda i: (i, 0)),
        out_shape=jax.ShapeDtypeStruct((n_edges, out_ch // 2), jnp.float32),
    )(zstats, W_edge, be, ge, betae, w3, bm, edge_attr)

    info = plsc.get_sparse_core_info()
    nc, ns = info.num_cores, info.num_subcores
    nw = nc * ns
    row = edge_index[0]
    col = edge_index[1]
    zeros_tab = jnp.zeros((n_pad, 128), jnp.float32)
    ones_rows = jnp.ones((80, 128), jnp.float32)
    deg_k = _make_deg_kernel(n_pad, n_edges, nc, ns)
    d_part = deg_k(col, zeros_tab, ones_rows)
    sc = _make_sc_kernel(n_pad, n_edges, nc, ns)
    s_part, ystats = sc(a_tab, b_tab, p_tab, row, col, zeros_tab)

    dgrp = n_pad // 128
    s_part4 = s_part.reshape(nc, dgrp, 128, 128)
    d_part4 = d_part.reshape(nc, dgrp, 128, 128)
    nf4 = nf.reshape(dgrp, 128, 128)
    gblk = 16
    g3 = dgrp // gblk
    out = pl.pallas_call(
        functools.partial(_epilogue, n_edges),
        grid=(g3,),
        in_specs=[
            pl.BlockSpec((nw, 2, out_ch), lambda i: (0, 0, 0)),
            pl.BlockSpec((1, out_ch), lambda i: (0, 0)),
            pl.BlockSpec((1, out_ch), lambda i: (0, 0)),
            pl.BlockSpec((nc, gblk, 128, 128), lambda i: (0, i, 0, 0)),
            pl.BlockSpec((nc, gblk, 128, 128), lambda i: (0, i, 0, 0)),
            pl.BlockSpec((gblk, 128, 128), lambda i: (i, 0, 0)),
        ],
        out_specs=pl.BlockSpec((gblk, 128, 128), lambda i: (i, 0, 0)),
        out_shape=jax.ShapeDtypeStruct((dgrp, 128, 128), jnp.float32),
    )(ystats, gm, betam, s_part4, d_part4, nf4)
    return out.reshape(n_pad, out_ch)[:n_nodes]
